# baseline (speedup 1.0000x reference)
import functools

import jax
import jax.numpy as jnp
from jax import lax
from jax.experimental import pallas as pl
from jax.experimental.pallas import tpu as pltpu
from jax.experimental.pallas import tpu_sc as plsc

_WAVELENGTH_MIN = 0.01
_WAVELENGTH_MAX = 0.5
_FOCAL_DEPTH = 0.5

_L = 16
_NC = 2
_NS = 16
_NW = _NC * _NS
_COLS = 512
_ROWS = 64 * 512
_ROWS_W = _ROWS // _NW
_CHUNK_R = 32
_NCH = _ROWS_W // _CHUNK_R
_NSLOT = 3
_NSP = 2


def _body(depth_hbm, w_hbm, out_hbm, wv, sp, *refs):
    bufs = refs[0:_NSLOT]
    isems = refs[_NSLOT:2 * _NSLOT]
    stsem = refs[2 * _NSLOT]
    wsem = refs[2 * _NSLOT + 1]

    c = lax.axis_index("c")
    s = lax.axis_index("s")
    wid = s * _NC + c
    base = wid * _ROWS_W

    pltpu.sync_copy(w_hbm, wv)
    lam = jnp.clip(jnp.abs(wv[...]), _WAVELENGTH_MIN, _WAVELENGTH_MAX)
    scale = (2.0 * jnp.pi) / lam

    def in_cp(k, b):
        start = pl.multiple_of(base + k * _CHUNK_R, _CHUNK_R)
        return pltpu.make_async_copy(
            depth_hbm.at[pl.ds(start, _CHUNK_R), :], bufs[b], isems[b])

    def stage_cp(k, b):
        q = lax.rem(k, _NSP)
        return pltpu.make_async_copy(bufs[b], sp.at[s, q], stsem.at[q])

    def write_cp(k):
        q = lax.rem(k, _NSP)
        start = pl.multiple_of(base + k * _CHUNK_R, _CHUNK_R)
        return pltpu.make_async_copy(
            sp.at[s, q], out_hbm.at[pl.ds(start, _CHUNK_R), :], wsem.at[q])

    def compute(buf):
        @plsc.parallel_loop(0, _CHUNK_R, unroll=2)
        def _(r):
            for j in range(_COLS // _L):
                x = buf[r, pl.ds(j * _L, _L)]
                buf[r, pl.ds(j * _L, _L)] = scale * jnp.abs(x - _FOCAL_DEPTH)

    in_cp(0, 0).start()
    in_cp(1, 1).start()

    def step(t, carry):
        for b in range(_NSLOT):
            k = _NSLOT * t + b
            in_cp(k, b).wait()
            compute(bufs[b])

            if b == 0 or b == 1:
                @pl.when(t > 0)
                def _():
                    write_cp(k - _NSP).wait()
            else:
                write_cp(k - _NSP).wait()

            stage_cp(k, b).start()

            pb = (b + 2) % _NSLOT
            if b == 0:
                @pl.when(t > 0)
                def _():
                    stage_cp(k - 1, pb).wait()
                    write_cp(k - 1).start()

                in_cp(k + 2, pb).start()
            elif b == 1:
                stage_cp(k - 1, pb).wait()
                write_cp(k - 1).start()
                in_cp(k + 2, pb).start()
            else:
                stage_cp(k - 1, pb).wait()
                write_cp(k - 1).start()

                @pl.when(t + 1 < _NCH // _NSLOT)
                def _():
                    in_cp(k + 2, pb).start()
        return carry

    lax.fori_loop(0, _NCH // _NSLOT, step, 0)

    in_cp(_NCH - 1, 1).start()

    in_cp(_NCH - 2, 0).wait()
    compute(bufs[0])
    write_cp(_NCH - 4).wait()
    stage_cp(_NCH - 2, 0).start()
    stage_cp(_NCH - 3, 2).wait()
    write_cp(_NCH - 3).start()

    in_cp(_NCH - 1, 1).wait()
    compute(bufs[1])
    write_cp(_NCH - 3).wait()
    stage_cp(_NCH - 1, 1).start()
    stage_cp(_NCH - 2, 0).wait()
    write_cp(_NCH - 2).start()
    stage_cp(_NCH - 1, 1).wait()
    write_cp(_NCH - 1).start()

    write_cp(_NCH - 2).wait()
    write_cp(_NCH - 1).wait()


@functools.partial(jax.jit, static_argnames=())
def kernel(depth, w_raw):
    w16 = jnp.broadcast_to(jnp.asarray(w_raw, jnp.float32), (_L,))
    rows = depth.reshape(_ROWS, _COLS)
    mesh = plsc.VectorSubcoreMesh(core_axis_name="c", subcore_axis_name="s")
    run = pl.kernel(
        _body,
        out_type=jax.ShapeDtypeStruct((_ROWS, _COLS), jnp.float32),
        mesh=mesh,
        compiler_params=pltpu.CompilerParams(use_tc_tiling_on_sc=True),
        scratch_types=(
            [pltpu.VMEM((_L,), jnp.float32)]
            + [pltpu.VMEM_SHARED((_NS, _NSP, _CHUNK_R, _COLS), jnp.float32)]
            + [pltpu.VMEM((_CHUNK_R, _COLS), jnp.float32)] * _NSLOT
            + [pltpu.SemaphoreType.DMA] * _NSLOT
            + [pltpu.SemaphoreType.DMA((_NSP,))] * 2
        ),
    )
    out = run(rows, w16)
    return out.reshape(depth.shape)


# speedup vs baseline: 1.1270x; 1.1270x over previous
import functools

import jax
import jax.numpy as jnp
from jax import lax
from jax.experimental import pallas as pl
from jax.experimental.pallas import tpu as pltpu
from jax.experimental.pallas import tpu_sc as plsc

_WAVELENGTH_MIN = 0.01
_WAVELENGTH_MAX = 0.5
_FOCAL_DEPTH = 0.5

_L = 16
_NC = 2
_NS = 16
_NW = _NC * _NS
_COLS = 512
_ROWS = 64 * 512
_ROWS_W = _ROWS // _NW
_CHUNK_R = 64
_NCH = _ROWS_W // _CHUNK_R
_NSLOT = 3


def _body(depth_hbm, w_hbm, out_hbm, wv, *refs):
    bufs = refs[0:_NSLOT]
    isems = refs[_NSLOT:2 * _NSLOT]
    osems = refs[2 * _NSLOT:3 * _NSLOT]

    c = lax.axis_index("c")
    s = lax.axis_index("s")
    wid = s * _NC + c
    base = wid * _ROWS_W

    pltpu.sync_copy(w_hbm, wv)
    lam = jnp.clip(jnp.abs(wv[...]), _WAVELENGTH_MIN, _WAVELENGTH_MAX)
    scale = (2.0 * jnp.pi) / lam

    def in_cp(k, b):
        start = pl.multiple_of(base + k * _CHUNK_R, _CHUNK_R)
        return pltpu.make_async_copy(
            depth_hbm.at[pl.ds(start, _CHUNK_R), :], bufs[b], isems[b])

    def out_cp(k, b):
        start = pl.multiple_of(base + k * _CHUNK_R, _CHUNK_R)
        return pltpu.make_async_copy(
            bufs[b], out_hbm.at[pl.ds(start, _CHUNK_R), :], osems[b])

    def compute(buf):
        @plsc.parallel_loop(0, _CHUNK_R, unroll=2)
        def _(r):
            for j in range(_COLS // _L):
                x = buf[r, pl.ds(j * _L, _L)]
                buf[r, pl.ds(j * _L, _L)] = scale * jnp.abs(x - _FOCAL_DEPTH)

    in_cp(0, 0).start()
    in_cp(1, 1).start()

    def process(k, b):
        in_cp(k, b).wait()
        compute(bufs[b])
        out_cp(k, b).start()

    def drain(k, b):
        out_cp(k, b).wait()

    def step(t, carry):
        for b in range(_NSLOT):
            k = _NSLOT * t + b
            process(k, b)
            nb = (b + 2) % _NSLOT
            if b == 0:
                @pl.when(t > 0)
                def _():
                    drain(k - 1, nb)

                in_cp(k + 2, nb).start()
            elif b == 1:
                drain(k - 1, nb)
                in_cp(k + 2, nb).start()
            else:
                @pl.when(t + 1 < _NCH // _NSLOT)
                def _():
                    drain(k - 1, nb)
                    in_cp(k + 2, nb).start()
        return carry

    lax.fori_loop(0, _NCH // _NSLOT, step, 0)

    process(_NCH - 1, 0)

    drain(_NCH - 3, 1)
    drain(_NCH - 2, 2)
    drain(_NCH - 1, 0)


@functools.partial(jax.jit, static_argnames=())
def kernel(depth, w_raw):
    w16 = jnp.broadcast_to(jnp.asarray(w_raw, jnp.float32), (_L,))
    rows = depth.reshape(_ROWS, _COLS)
    mesh = plsc.VectorSubcoreMesh(core_axis_name="c", subcore_axis_name="s")
    run = pl.kernel(
        _body,
        out_type=jax.ShapeDtypeStruct((_ROWS, _COLS), jnp.float32),
        mesh=mesh,
        compiler_params=pltpu.CompilerParams(use_tc_tiling_on_sc=True),
        scratch_types=(
            [pltpu.VMEM((_L,), jnp.float32)]
            + [pltpu.VMEM((_CHUNK_R, _COLS), jnp.float32)] * _NSLOT
            + [pltpu.SemaphoreType.DMA] * (2 * _NSLOT)
        ),
    )
    out = run(rows, w16)
    return out.reshape(depth.shape)
